# baseline (speedup 1.0000x reference)
import functools

import jax
import jax.numpy as jnp
from jax import lax
from jax.experimental import pallas as pl
from jax.experimental.pallas import tpu as pltpu
from jax.experimental.pallas import tpu_sc as plsc

NC = 2
NS = 16
NW = NC * NS
LANES = 16
EPS = 1e-5


def _rsqrt(v):
    i = lax.bitcast_convert_type(v, jnp.int32)
    i = jnp.int32(0x5F3759DF) - lax.shift_right_logical(i, 1)
    y = lax.bitcast_convert_type(i, jnp.float32)
    for _ in range(3):
        y = y * (1.5 - 0.5 * v * y * y)
    return y


def _make_sc_kernel(T, L, D, CH):
    tpw = T // NW
    n_chunks = tpw // CH
    nd = D // LANES

    mesh = plsc.VectorSubcoreMesh(core_axis_name="c", subcore_axis_name="s")

    @functools.partial(
        pl.kernel,
        out_type=jax.ShapeDtypeStruct((T, D), jnp.float32),
        mesh=mesh,
        compiler_params=pltpu.CompilerParams(
            needs_layout_passes=False, use_tc_tiling_on_sc=False),
        scratch_types=[
            pltpu.VMEM((tpw,), jnp.int32),
            pltpu.VMEM((CH, D), jnp.float32),
            pltpu.VMEM((2, D), jnp.float32),
            pltpu.VMEM((CH, D), jnp.float32),
            pltpu.SemaphoreType.DMA,
        ],
    )
    def sc_kernel(ids_hbm, pos_hbm, gb_hbm, table_hbm, out_hbm,
                  idx_v, pos_v, gb_v, rows_v, sem):
        wid = lax.axis_index("s") * NC + lax.axis_index("c")
        base = wid * tpw

        pltpu.sync_copy(ids_hbm.at[pl.ds(base, tpw)], idx_v)
        for r in range(CH // L):
            pltpu.sync_copy(pos_hbm.at[pl.ds(0, L)],
                            pos_v.at[pl.ds(r * L, L)])
        pltpu.sync_copy(gb_hbm, gb_v)

        g = [gb_v[0, d * LANES:(d + 1) * LANES] for d in range(nd)]
        b = [gb_v[1, d * LANES:(d + 1) * LANES] for d in range(nd)]
        inv_d = jnp.float32(1.0 / D)

        def hsum(v):
            return jnp.full((LANES,), jnp.sum(v), jnp.float32)

        def token_body(t, _):
            x = [rows_v[t, d * LANES:(d + 1) * LANES]
                 + pos_v[t, d * LANES:(d + 1) * LANES] for d in range(nd)]
            s = x[0]
            q = x[0] * x[0]
            for d in range(1, nd):
                s = s + x[d]
                q = q + x[d] * x[d]
            mv = hsum(s) * inv_d
            var = hsum(q) * inv_d - mv * mv + jnp.float32(EPS)
            rv = _rsqrt(var)
            for d in range(nd):
                rows_v[t, d * LANES:(d + 1) * LANES] = (
                    (x[d] - mv) * rv * g[d] + b[d])
            return 0

        def chunk_body(c, _):
            row0 = base + c * CH
            pltpu.async_copy(
                table_hbm.at[idx_v.at[pl.ds(c * CH, CH)]], rows_v, sem
            ).wait()
            lax.fori_loop(0, CH, token_body, 0)
            pltpu.sync_copy(rows_v, out_hbm.at[pl.ds(row0, CH)])
            return 0

        lax.fori_loop(0, n_chunks, chunk_body, 0)

    return sc_kernel


def kernel(input_ids, emb_table, pos_table, ln_gamma, ln_beta):
    B, L = input_ids.shape
    D = emb_table.shape[1]
    T = B * L
    CH = 2 * L
    assert T % NW == 0 and (T // NW) % CH == 0 and D % LANES == 0
    assert (T // NW) % 8 == 0 and CH % 8 == 0

    ids_flat = input_ids.reshape(T).astype(jnp.int32)
    gb = jnp.stack([ln_gamma, ln_beta]).astype(jnp.float32)
    sc = _make_sc_kernel(T, L, D, CH)
    out = sc(ids_flat, pos_table, gb, emb_table)
    return out.reshape(B, L, D)


# speedup vs baseline: 1.1690x; 1.1690x over previous
import functools

import jax
import jax.numpy as jnp
from jax import lax
from jax.experimental import pallas as pl
from jax.experimental.pallas import tpu as pltpu
from jax.experimental.pallas import tpu_sc as plsc

NC = 2
NS = 16
NW = NC * NS
LANES = 16
EPS = 1e-5


def _rsqrt(v):
    i = lax.bitcast_convert_type(v, jnp.int32)
    i = jnp.int32(0x5F3759DF) - lax.shift_right_logical(i, 1)
    y = lax.bitcast_convert_type(i, jnp.float32)
    for _ in range(3):
        y = y * (1.5 - 0.5 * v * y * y)
    return y


def _make_sc_kernel(T, L, D, CH):
    tpw = T // NW
    n_chunks = tpw // CH
    nd = D // LANES

    mesh = plsc.VectorSubcoreMesh(core_axis_name="c", subcore_axis_name="s")

    @functools.partial(
        pl.kernel,
        out_type=jax.ShapeDtypeStruct((T, D), jnp.float32),
        mesh=mesh,
        compiler_params=pltpu.CompilerParams(
            needs_layout_passes=False, use_tc_tiling_on_sc=False),
        scratch_types=[
            pltpu.VMEM((tpw,), jnp.int32),
            pltpu.VMEM((CH, D), jnp.float32),
            pltpu.VMEM((2, D), jnp.float32),
            pltpu.VMEM((CH, D), jnp.float32),
            pltpu.VMEM((CH, D), jnp.float32),
            pltpu.SemaphoreType.DMA,
            pltpu.SemaphoreType.DMA,
        ],
    )
    def sc_kernel(ids_hbm, pos_hbm, gb_hbm, table_hbm, out_hbm,
                  idx_v, pos_v, gb_v, rows0_v, rows1_v, sem0, sem1):
        wid = lax.axis_index("s") * NC + lax.axis_index("c")
        base = wid * tpw

        pltpu.sync_copy(ids_hbm.at[pl.ds(base, tpw)], idx_v)
        for r in range(CH // L):
            pltpu.sync_copy(pos_hbm.at[pl.ds(0, L)],
                            pos_v.at[pl.ds(r * L, L)])
        pltpu.sync_copy(gb_hbm, gb_v)

        g = [gb_v[0, d * LANES:(d + 1) * LANES] for d in range(nd)]
        b = [gb_v[1, d * LANES:(d + 1) * LANES] for d in range(nd)]
        inv_d = jnp.float32(1.0 / D)

        def hsum(v):
            return jnp.full((LANES,), jnp.sum(v), jnp.float32)

        def make_token_body(rows_v):
            def token_body(t):
                x = [rows_v[t, d * LANES:(d + 1) * LANES]
                     + pos_v[t, d * LANES:(d + 1) * LANES] for d in range(nd)]
                s = x[0]
                q = x[0] * x[0]
                for d in range(1, nd):
                    s = s + x[d]
                    q = q + x[d] * x[d]
                mv = hsum(s) * inv_d
                var = hsum(q) * inv_d - mv * mv + jnp.float32(EPS)
                rv = _rsqrt(var)
                for d in range(nd):
                    rows_v[t, d * LANES:(d + 1) * LANES] = (
                        (x[d] - mv) * rv * g[d] + b[d])
            return token_body

        bufs = [(rows0_v, sem0), (rows1_v, sem1)]

        def gather_start(c, buf, sem):
            return pltpu.async_copy(
                table_hbm.at[idx_v.at[pl.ds(c * CH, CH)]], buf, sem)

        gather_start(0, rows0_v, sem0)

        def chunk2_body(c2, _):
            for p in range(2):
                c = c2 * 2 + p
                buf, sem = bufs[p]
                nbuf, nsem = bufs[1 - p]

                @pl.when(c + 1 < n_chunks)
                def _():
                    gather_start(c + 1, nbuf, nsem)

                pltpu.make_async_copy(
                    table_hbm.at[idx_v.at[pl.ds(c * CH, CH)]], buf, sem
                ).wait()
                plsc.parallel_loop(0, CH, unroll=8)(make_token_body(buf))
                pltpu.sync_copy(buf, out_hbm.at[pl.ds(base + c * CH, CH)])
            return 0

        lax.fori_loop(0, n_chunks // 2, chunk2_body, 0)

    return sc_kernel


def kernel(input_ids, emb_table, pos_table, ln_gamma, ln_beta):
    B, L = input_ids.shape
    D = emb_table.shape[1]
    T = B * L
    CH = 2 * L
    assert T % NW == 0 and (T // NW) % CH == 0 and D % LANES == 0
    assert (T // NW) % 8 == 0 and CH % 8 == 0

    ids_flat = input_ids.reshape(T).astype(jnp.int32)
    gb = jnp.stack([ln_gamma, ln_beta]).astype(jnp.float32)
    sc = _make_sc_kernel(T, L, D, CH)
    out = sc(ids_flat, pos_table, gb, emb_table)
    return out.reshape(B, L, D)


# speedup vs baseline: 1.5131x; 1.2944x over previous
import functools

import jax
import jax.numpy as jnp
from jax import lax
from jax.experimental import pallas as pl
from jax.experimental.pallas import tpu as pltpu
from jax.experimental.pallas import tpu_sc as plsc

NC = 2
NS = 16
NW = NC * NS
LANES = 16
EPS = 1e-5
BB = 128
STRIDE = 129


def _rsqrt(v):
    i = lax.bitcast_convert_type(v, jnp.int32)
    i = jnp.int32(0x5F3759DF) - lax.shift_right_logical(i, 1)
    y = lax.bitcast_convert_type(i, jnp.float32)
    for _ in range(3):
        y = y * (1.5 - 0.5 * v * y * y)
    return y


def _make_sc_kernel(B, L, D):
    T = B * L
    n_blocks = T // BB
    bpw = n_blocks // NW
    nd = D // LANES
    ncol = BB // LANES
    dt_n = D // 8
    bt_n = B // BB

    mesh = plsc.VectorSubcoreMesh(core_axis_name="c", subcore_axis_name="s")

    @functools.partial(
        pl.kernel,
        out_type=jax.ShapeDtypeStruct((L, dt_n, bt_n, 8, BB), jnp.float32),
        mesh=mesh,
        compiler_params=pltpu.CompilerParams(
            needs_layout_passes=False, use_tc_tiling_on_sc=False),
        scratch_types=[
            pltpu.VMEM((bpw * BB,), jnp.int32),
            pltpu.VMEM((L, D), jnp.float32),
            pltpu.VMEM((2, D, LANES), jnp.float32),
            pltpu.VMEM((BB, D), jnp.float32),
            pltpu.VMEM((BB, D), jnp.float32),
            pltpu.VMEM((D * STRIDE,), jnp.float32),
            pltpu.VMEM((D, BB), jnp.float32),
            pltpu.VMEM((D, BB), jnp.float32),
            pltpu.SemaphoreType.DMA,
            pltpu.SemaphoreType.DMA,
            pltpu.SemaphoreType.DMA,
            pltpu.SemaphoreType.DMA,
        ],
    )
    def sc_kernel(ids_hbm, pos_hbm, gb_hbm, table_hbm, out_hbm,
                  idx_v, pos_v, gb_v, g0_v, g1_v, tt_v, s0_v, s1_v,
                  gsem0, gsem1, ssem0, ssem1):
        wid = lax.axis_index("s") * NC + lax.axis_index("c")
        g_base = wid * bpw

        pltpu.sync_copy(ids_hbm.at[pl.ds(g_base * BB, bpw * BB)], idx_v)
        pltpu.sync_copy(pos_hbm.at[pl.ds(0, L)], pos_v)
        pltpu.sync_copy(gb_hbm, gb_v)

        lane = lax.iota(jnp.int32, LANES)
        inv_d = jnp.float32(1.0 / D)
        sc_idx = [(lane + dq * LANES) * jnp.int32(STRIDE) for dq in range(nd)]
        ld_idx = [lane + jnp.int32(c * LANES) for c in range(ncol)]

        gbufs = [(g0_v, gsem0), (g1_v, gsem1)]
        sbufs = [(s0_v, ssem0), (s1_v, ssem1)]

        def gather_start(i, buf, sem):
            return pltpu.async_copy(
                table_hbm.at[idx_v.at[pl.ds(i * BB, BB)]], buf, sem)

        def block_lbt(i):
            g = g_base + i
            return g // bt_n, g % bt_n

        def out_store(i, buf, sem, do_wait):
            l, bt = block_lbt(i)
            for dt in range(dt_n):
                cp = pltpu.make_async_copy(
                    buf.at[pl.ds(dt * 8, 8)], out_hbm.at[l, dt, bt], sem)
                if do_wait:
                    cp.wait()
                else:
                    cp.start()

        def compute(i, gbuf, sbuf):
            l, _ = block_lbt(i)
            pvec = [pos_v[l, dq * LANES:(dq + 1) * LANES] for dq in range(nd)]

            @plsc.parallel_loop(0, BB, unroll=4)
            def _(t):
                for dq in range(nd):
                    x = gbuf[t, dq * LANES:(dq + 1) * LANES] + pvec[dq]
                    plsc.store_scatter(tt_v, [sc_idx[dq] + t], x)

            def stats(d, carry):
                s, q = carry
                base = d * jnp.int32(STRIDE)
                s2, q2 = [], []
                for c in range(ncol):
                    v = plsc.load_gather(tt_v, [ld_idx[c] + base])
                    s2.append(s[c] + v)
                    q2.append(q[c] + v * v)
                return s2, q2

            zero = [jnp.zeros((LANES,), jnp.float32)] * ncol
            s, q = plsc.parallel_loop(0, D, carry=(zero, zero), unroll=4)(
                stats)

            mv, rv = [], []
            for c in range(ncol):
                m = s[c] * inv_d
                var = q[c] * inv_d - m * m + jnp.float32(EPS)
                mv.append(m)
                rv.append(_rsqrt(var))

            @plsc.parallel_loop(0, D, unroll=4)
            def _(d):
                base = d * jnp.int32(STRIDE)
                gd = gb_v[0, d]
                bd = gb_v[1, d]
                for c in range(ncol):
                    v = plsc.load_gather(tt_v, [ld_idx[c] + base])
                    sbuf[d, c * LANES:(c + 1) * LANES] = (
                        (v - mv[c]) * rv[c] * gd + bd)

        gather_start(0, g0_v, gsem0)

        def pair_body(i2, _):
            for p in range(2):
                i = i2 * 2 + p
                gbuf, gsem = gbufs[p]
                sbuf, ssem = sbufs[p]
                ngbuf, ngsem = gbufs[1 - p]

                @pl.when(i + 1 < bpw)
                def _():
                    gather_start(i + 1, ngbuf, ngsem)

                pltpu.make_async_copy(
                    table_hbm.at[idx_v.at[pl.ds(i * BB, BB)]], gbuf, gsem
                ).wait()

                @pl.when(i >= 2)
                def _():
                    out_store(i - 2, sbuf, ssem, do_wait=True)

                compute(i, gbuf, sbuf)
                out_store(i, sbuf, ssem, do_wait=False)
            return 0

        lax.fori_loop(0, bpw // 2, pair_body, 0)
        for p in range(2):
            sbuf, ssem = sbufs[p]
            out_store(bpw - 2 + p, sbuf, ssem, do_wait=True)

    return sc_kernel


def kernel(input_ids, emb_table, pos_table, ln_gamma, ln_beta):
    B, L = input_ids.shape
    D = emb_table.shape[1]
    T = B * L
    assert T % (NW * BB) == 0 and B % BB == 0 and D % LANES == 0
    assert (T // (NW * BB)) % 2 == 0

    ids_lm = jnp.transpose(input_ids, (1, 0)).reshape(T).astype(jnp.int32)
    gb = jnp.stack([ln_gamma, ln_beta]).astype(jnp.float32)
    gb = jnp.broadcast_to(gb[:, :, None], (2, D, LANES))
    sc = _make_sc_kernel(B, L, D)
    out5 = sc(ids_lm, pos_table.astype(jnp.float32)[:L], gb, emb_table)
    return out5.transpose(2, 4, 0, 1, 3).reshape(B, L, D)
